# baseline (speedup 1.0000x reference)
import math

import jax
import jax.numpy as jnp
from jax import lax
from jax.experimental import pallas as pl
from jax.experimental.pallas import tpu as pltpu
from jax.experimental.pallas import tpu_sc as plsc

_VOCAB = 1000000
_D = 64
_B = 4096 * 200
_NC = 2
_NS = 16
_NW = _NC * _NS
_R = 128
_CPW = _B // (_NW * _R)
_NBUF = 4
_NOUTER = _CPW // _NBUF
_SCALE = math.sqrt(_D)
_LANES = 16


def _scband_emb_body(x_hbm, lut_hbm, out_hbm, idx_all, gbuf, obuf, gsem, ssem):
    cid = lax.axis_index("c")
    sid = lax.axis_index("s")
    wid = sid * _NC + cid
    chunk0 = wid * _CPW
    row0 = wid * _CPW * _R

    pltpu.sync_copy(x_hbm.at[pl.ds(chunk0, _CPW)], idx_all)

    def start_gather(b, c):
        pltpu.async_copy(lut_hbm.at[idx_all.at[c]], gbuf.at[b], gsem.at[b])

    def wait_gather(b):
        pltpu.make_async_copy(
            lut_hbm.at[idx_all.at[0]], gbuf.at[b], gsem.at[b]
        ).wait()

    def start_scatter(b, c):
        pltpu.async_copy(
            obuf.at[b], out_hbm.at[pl.ds(row0 + c * _R, _R)], ssem.at[b]
        )

    def wait_scatter(b):
        pltpu.make_async_copy(
            obuf.at[b], out_hbm.at[pl.ds(row0, _R)], ssem.at[b]
        ).wait()

    for b in range(_NBUF):
        start_gather(b, b)

    def outer(g, carry):
        for b in range(_NBUF):
            c = g * _NBUF + b
            wait_gather(b)

            @pl.when(g > 0)
            def _():
                wait_scatter(b)

            src = gbuf.at[b]
            dst = obuf.at[b]

            @plsc.parallel_loop(0, _R, step=1, unroll=8)
            def _(i):
                for j in range(_D // _LANES):
                    sl = pl.ds(j * _LANES, _LANES)
                    dst[i, sl] = src[i, sl] * _SCALE

            start_scatter(b, c)

            @pl.when(g < _NOUTER - 1)
            def _():
                start_gather(b, c + _NBUF)
        return carry

    lax.fori_loop(0, _NOUTER, outer, 0)

    for b in range(_NBUF):
        wait_scatter(b)


def kernel(x, lut):
    x_flat = x.reshape(_B // _R, _R).astype(jnp.int32)
    mesh = plsc.VectorSubcoreMesh(
        core_axis_name="c", subcore_axis_name="s",
        num_cores=_NC, num_subcores=_NS,
    )
    emb = pl.kernel(
        _scband_emb_body,
        out_type=jax.ShapeDtypeStruct((_B, _D), jnp.float32),
        mesh=mesh,
        compiler_params=pltpu.CompilerParams(use_tc_tiling_on_sc=False),
        scratch_types=[
            pltpu.VMEM((_CPW, _R), jnp.int32),
            pltpu.VMEM((_NBUF, _R, _D), jnp.float32),
            pltpu.VMEM((_NBUF, _R, _D), jnp.float32),
            pltpu.SemaphoreType.DMA((_NBUF,)),
            pltpu.SemaphoreType.DMA((_NBUF,)),
        ],
    )(x_flat, lut)
    return emb.reshape(x.shape + (_D,))
